# baseline (speedup 1.0000x reference)
import functools

import jax
import jax.numpy as jnp
from jax import lax
from jax.experimental import pallas as pl
from jax.experimental.pallas import tpu as pltpu
from jax.experimental.pallas import tpu_sc as plsc


def _make_repack_kernel(V, D):
    info = plsc.get_sparse_core_info()
    NC, NS, LN = info.num_cores, info.num_subcores, info.num_lanes
    NW = NC * NS
    CW = 1024
    OR = CW * D // 128
    NFULL = V // CW
    TAILW = V - NFULL * CW
    assert TAILW > 0 and TAILW % 64 == 0
    PERW = (NFULL + NW - 1) // NW
    mesh = plsc.VectorSubcoreMesh(core_axis_name="c", subcore_axis_name="s")
    g_per_row = 128 // LN
    TPR = 128 // D
    UNR = 4
    assert OR % UNR == 0 and (TAILW * D // 128) % UNR == 0

    @functools.partial(
        pl.kernel,
        out_type=jax.ShapeDtypeStruct((V * D // 128, 128), jnp.float32),
        mesh=mesh,
        compiler_params=pltpu.CompilerParams(
            needs_layout_passes=False, use_tc_tiling_on_sc=True,
            disable_bounds_checks=True),
        scratch_types=[
            pltpu.VMEM((D, CW), jnp.float32),
            pltpu.VMEM((OR, 128), jnp.float32),
            pltpu.VMEM((TAILW * D // 128, 128), jnp.float32),
            pltpu.SemaphoreType.DMA,
        ],
    )
    def repack(tab_h, tail_h, out_h, blk_v, out_v, tail_v, sem):
        wid = lax.axis_index("s") * NC + lax.axis_index("c")
        d_base = [jnp.arange(LN, dtype=jnp.int32) + LN * (g % (D // LN))
                  for g in range(g_per_row)]

        iota = jnp.arange(LN, dtype=jnp.int32)

        def transpose_block(rows):
            n_grp = rows * TPR // LN
            for h in range(D // LN):
                d_const = LN * h + iota
                col_const = (iota % TPR) * D + LN * h + iota
                row_const = iota // TPR

                @plsc.parallel_loop(0, n_grp, unroll=4)
                def _(k):
                    t_vec = jnp.full((LN,), k * LN, jnp.int32) + iota
                    row_vec = jnp.full(
                        (LN,), k * (LN // TPR), jnp.int32) + row_const
                    v = plsc.load_gather(blk_v, [d_const, t_vec])
                    plsc.store_scatter(out_v, [row_vec, col_const], v)

        def chunk_body(s, _):
            cid = wid + NW * s

            @pl.when(cid < NFULL)
            def _():
                pltpu.sync_copy(tab_h.at[:, pl.ds(cid * CW, CW)], blk_v)
                transpose_block(OR)
                r0 = pl.multiple_of(cid * OR, 8)
                pltpu.sync_copy(out_v, out_h.at[pl.ds(r0, OR)])
            return 0

        lax.fori_loop(0, PERW, chunk_body, 0)

        @pl.when(wid == NW - 1)
        def _():
            pltpu.sync_copy(tail_h, tail_v)
            pltpu.sync_copy(
                tail_v, out_h.at[pl.ds(NFULL * OR, TAILW * D // 128)])

    return repack, TAILW


def _make_pool_kernel(B, L, D):
    info = plsc.get_sparse_core_info()
    NC, NS, LN = info.num_cores, info.num_subcores, info.num_lanes
    NW = NC * NS
    assert B % NW == 0 and D == 2 * LN and L % 8 == 0
    BPW = B // NW
    CH1 = 128
    CH2 = L - CH1
    mesh = plsc.VectorSubcoreMesh(core_axis_name="c", subcore_axis_name="s")

    @functools.partial(
        pl.kernel,
        out_type=jax.ShapeDtypeStruct((B, 3 * D), jnp.float32),
        mesh=mesh,
        compiler_params=pltpu.CompilerParams(
            needs_layout_passes=False, use_tc_tiling_on_sc=False,
            disable_bounds_checks=True),
        scratch_types=[
            pltpu.VMEM((BPW, L), jnp.int32),
            pltpu.VMEM((BPW,), jnp.int32),
            pltpu.VMEM((L, D), jnp.float32),
            pltpu.VMEM((BPW, 3 * D), jnp.float32),
            pltpu.SemaphoreType.DMA,
        ],
    )
    def pool(x_h, len_h, tab_h, rep_h, idx_v, len_v, rows_v, rep_v, sem):
        wid = lax.axis_index("s") * NC + lax.axis_index("c")
        base = wid * BPW
        pltpu.sync_copy(x_h.at[pl.ds(base, BPW)], idx_v)
        pltpu.sync_copy(len_h.at[pl.ds(base, BPW)], len_v)

        def seq_body(i, _):
            pltpu.async_copy(
                tab_h.at[idx_v.at[i, pl.ds(0, CH1)]],
                rows_v.at[pl.ds(0, CH1)], sem).wait()
            pltpu.async_copy(
                tab_h.at[idx_v.at[i, pl.ds(CH1, CH2)]],
                rows_v.at[pl.ds(CH1, CH2)], sem).wait()
            g16 = pl.multiple_of((i // LN) * LN, 8)
            lvec = len_v[pl.ds(g16, LN)]
            lb = lax.gather(
                lvec,
                jnp.full((LN, 1), i % LN, jnp.int32),
                lax.GatherDimensionNumbers(
                    offset_dims=(), collapsed_slice_dims=(0,),
                    start_index_map=(0,)),
                slice_sizes=(1,),
                mode=lax.GatherScatterMode.PROMISE_IN_BOUNDS)
            n_rows = jnp.max(lb)
            inv_len = 1.0 / lb.astype(jnp.float32)

            def row_body(j, c):
                mn0, mn1, mx0, mx1, s0, s1 = c
                r0 = rows_v[j, pl.ds(0, LN)]
                r1 = rows_v[j, pl.ds(LN, LN)]
                return (jnp.minimum(mn0, r0), jnp.minimum(mn1, r1),
                        jnp.maximum(mx0, r0), jnp.maximum(mx1, r1),
                        s0 + r0, s1 + r1)

            big = jnp.full((LN,), 3.0e38, jnp.float32)
            zero = jnp.zeros((LN,), jnp.float32)
            mn0, mn1, mx0, mx1, s0, s1 = lax.fori_loop(
                0, n_rows, row_body, (big, big, -big, -big, zero, zero))
            rep_v[i, pl.ds(0, LN)] = mn0
            rep_v[i, pl.ds(LN, LN)] = mn1
            rep_v[i, pl.ds(2 * LN, LN)] = s0 * inv_len
            rep_v[i, pl.ds(3 * LN, LN)] = s1 * inv_len
            rep_v[i, pl.ds(4 * LN, LN)] = mx0
            rep_v[i, pl.ds(5 * LN, LN)] = mx1
            return 0

        lax.fori_loop(0, BPW, seq_body, 0)
        pltpu.sync_copy(rep_v, rep_h.at[pl.ds(base, BPW)])

    return pool


def _linear(rep, w_t, b2):
    B, K = rep.shape
    OUT = w_t.shape[1]
    BLK = 512

    def body(rep_ref, w_ref, b_ref, out_ref):
        out_ref[...] = jnp.dot(
            rep_ref[...], w_ref[...],
            preferred_element_type=jnp.float32) + b_ref[...]

    return pl.pallas_call(
        body,
        grid=(B // BLK,),
        in_specs=[
            pl.BlockSpec((BLK, K), lambda i: (i, 0)),
            pl.BlockSpec((K, OUT), lambda i: (0, 0)),
            pl.BlockSpec((1, OUT), lambda i: (0, 0)),
        ],
        out_specs=pl.BlockSpec((BLK, OUT), lambda i: (i, 0)),
        out_shape=jax.ShapeDtypeStruct((B, OUT), jnp.float32),
    )(rep, w_t, b2)


def kernel(x, lengths, table, W, b):
    B, L = x.shape
    V, D = table.shape
    x32 = x.astype(jnp.int32)
    lens = lengths.astype(jnp.int32)
    repack, tailw = _make_repack_kernel(V, D)
    tab_t = table.T
    tail_rep = table[V - tailw:].reshape(tailw * D // 128, 128)
    t_lin = repack(tab_t, tail_rep).reshape(V, D)
    pool = _make_pool_kernel(B, L, D)
    rep = pool(x32, lens, t_lin)
    return _linear(rep, W.T, b.reshape(1, -1))


# speedup vs baseline: 1.7668x; 1.7668x over previous
import functools

import jax
import jax.numpy as jnp
from jax import lax
from jax.experimental import pallas as pl
from jax.experimental.pallas import tpu as pltpu
from jax.experimental.pallas import tpu_sc as plsc


def _make_repack_kernel(V, D):
    info = plsc.get_sparse_core_info()
    NC, NS, LN = info.num_cores, info.num_subcores, info.num_lanes
    NW = NC * NS
    CW = 1024
    OR = CW * D // 128
    NFULL = V // CW
    TAILW = V - NFULL * CW
    assert TAILW > 0 and TAILW % 64 == 0
    PERW = (NFULL + NW - 1) // NW
    mesh = plsc.VectorSubcoreMesh(core_axis_name="c", subcore_axis_name="s")
    g_per_row = 128 // LN
    TPR = 128 // D
    UNR = 4
    assert OR % UNR == 0 and (TAILW * D // 128) % UNR == 0

    @functools.partial(
        pl.kernel,
        out_type=jax.ShapeDtypeStruct((V * D // 128, 128), jnp.float32),
        mesh=mesh,
        compiler_params=pltpu.CompilerParams(
            needs_layout_passes=False, use_tc_tiling_on_sc=True,
            disable_bounds_checks=True),
        scratch_types=[
            pltpu.VMEM((D, CW), jnp.float32),
            pltpu.VMEM((OR, 128), jnp.float32),
            pltpu.VMEM((TAILW * D // 128, 128), jnp.float32),
            pltpu.SemaphoreType.DMA,
        ],
    )
    def repack(tab_h, tail_h, out_h, blk_v, out_v, tail_v, sem):
        wid = lax.axis_index("s") * NC + lax.axis_index("c")
        d_base = [jnp.arange(LN, dtype=jnp.int32) + LN * (g % (D // LN))
                  for g in range(g_per_row)]

        iota = jnp.arange(LN, dtype=jnp.int32)

        def transpose_block(rows):
            n_grp = rows * TPR // LN
            for h in range(D // LN):
                d_const = LN * h + iota
                col_const = (iota % TPR) * D + LN * h + iota
                row_const = iota // TPR

                @plsc.parallel_loop(0, n_grp, unroll=4)
                def _(k):
                    t_vec = jnp.full((LN,), k * LN, jnp.int32) + iota
                    row_vec = jnp.full(
                        (LN,), k * (LN // TPR), jnp.int32) + row_const
                    v = plsc.load_gather(blk_v, [d_const, t_vec])
                    plsc.store_scatter(out_v, [row_vec, col_const], v)

        def chunk_body(s, _):
            cid = wid + NW * s

            @pl.when(cid < NFULL)
            def _():
                pltpu.sync_copy(tab_h.at[:, pl.ds(cid * CW, CW)], blk_v)
                transpose_block(OR)
                r0 = pl.multiple_of(cid * OR, 8)
                pltpu.sync_copy(out_v, out_h.at[pl.ds(r0, OR)])
            return 0

        lax.fori_loop(0, PERW, chunk_body, 0)

        @pl.when(wid == NW - 1)
        def _():
            pltpu.sync_copy(tail_h, tail_v)
            pltpu.sync_copy(
                tail_v, out_h.at[pl.ds(NFULL * OR, TAILW * D // 128)])

    return repack, TAILW


def _make_pool_kernel(B, L, D):
    info = plsc.get_sparse_core_info()
    NC, NS, LN = info.num_cores, info.num_subcores, info.num_lanes
    NW = NC * NS
    assert B % NW == 0 and D == 2 * LN and L % 8 == 0
    BPW = B // NW
    CH1 = 128
    CH2 = L - CH1
    mesh = plsc.VectorSubcoreMesh(core_axis_name="c", subcore_axis_name="s")

    @functools.partial(
        pl.kernel,
        out_type=jax.ShapeDtypeStruct((B, 3 * D), jnp.float32),
        mesh=mesh,
        compiler_params=pltpu.CompilerParams(
            needs_layout_passes=False, use_tc_tiling_on_sc=False,
            disable_bounds_checks=True),
        scratch_types=[
            pltpu.VMEM((BPW, L), jnp.int32),
            pltpu.VMEM((BPW,), jnp.int32),
            pltpu.VMEM((L, D), jnp.float32),
            pltpu.VMEM((L, D), jnp.float32),
            pltpu.VMEM((BPW, 3 * D), jnp.float32),
            pltpu.SemaphoreType.DMA,
            pltpu.SemaphoreType.DMA,
        ],
    )
    def pool(x_h, len_h, tab_h, rep_h, idx_v, len_v, rows0_v, rows1_v,
             rep_v, sem0, sem1):
        wid = lax.axis_index("s") * NC + lax.axis_index("c")
        base = wid * BPW
        pltpu.sync_copy(x_h.at[pl.ds(base, BPW)], idx_v)
        pltpu.sync_copy(len_h.at[pl.ds(base, BPW)], len_v)
        rows = (rows0_v, rows1_v)
        sems = (sem0, sem1)

        def start_gather(i, b):
            pltpu.async_copy(
                tab_h.at[idx_v.at[i, pl.ds(0, CH1)]],
                rows[b].at[pl.ds(0, CH1)], sems[b])
            pltpu.async_copy(
                tab_h.at[idx_v.at[i, pl.ds(CH1, CH2)]],
                rows[b].at[pl.ds(CH1, CH2)], sems[b])

        def wait_gather(i, b):
            pltpu.make_async_copy(
                tab_h.at[idx_v.at[i, pl.ds(0, CH1)]],
                rows[b].at[pl.ds(0, CH1)], sems[b]).wait()
            pltpu.make_async_copy(
                tab_h.at[idx_v.at[i, pl.ds(CH1, CH2)]],
                rows[b].at[pl.ds(CH1, CH2)], sems[b]).wait()

        def compute_seq(i, b):
            g16 = pl.multiple_of((i // LN) * LN, 8)
            lvec = len_v[pl.ds(g16, LN)]
            lb = lax.gather(
                lvec,
                jnp.full((LN, 1), i % LN, jnp.int32),
                lax.GatherDimensionNumbers(
                    offset_dims=(), collapsed_slice_dims=(0,),
                    start_index_map=(0,)),
                slice_sizes=(1,),
                mode=lax.GatherScatterMode.PROMISE_IN_BOUNDS)
            n_rows = jnp.max(lb)
            inv_len = 1.0 / lb.astype(jnp.float32)
            rows_v = rows[b]

            big = jnp.full((LN,), 3.0e38, jnp.float32)
            zero = jnp.zeros((LN,), jnp.float32)

            def row_body(j, c):
                mn0, mn1, mx0, mx1, s0, s1 = c
                r0 = rows_v[j, pl.ds(0, LN)]
                r1 = rows_v[j, pl.ds(LN, LN)]
                return (jnp.minimum(mn0, r0), jnp.minimum(mn1, r1),
                        jnp.maximum(mx0, r0), jnp.maximum(mx1, r1),
                        s0 + r0, s1 + r1)

            mn0, mn1, mx0, mx1, s0, s1 = plsc.parallel_loop(
                0, n_rows, unroll=4,
                carry=(big, big, -big, -big, zero, zero))(row_body)
            rep_v[i, pl.ds(0, LN)] = mn0
            rep_v[i, pl.ds(LN, LN)] = mn1
            rep_v[i, pl.ds(2 * LN, LN)] = s0 * inv_len
            rep_v[i, pl.ds(3 * LN, LN)] = s1 * inv_len
            rep_v[i, pl.ds(4 * LN, LN)] = mx0
            rep_v[i, pl.ds(5 * LN, LN)] = mx1

        start_gather(0, 0)
        start_gather(1, 1)

        def seq_body(p, _):
            for b in range(2):
                i = 2 * p + b
                wait_gather(i, b)
                compute_seq(i, b)

                @pl.when(i + 2 < BPW)
                def _(i=i, b=b):
                    start_gather(i + 2, b)
            return 0

        lax.fori_loop(0, BPW // 2, seq_body, 0)
        pltpu.sync_copy(rep_v, rep_h.at[pl.ds(base, BPW)])

    return pool


def _linear(rep, w_t, b2):
    B, K = rep.shape
    OUT = w_t.shape[1]
    BLK = 512

    def body(rep_ref, w_ref, b_ref, out_ref):
        out_ref[...] = jnp.dot(
            rep_ref[...], w_ref[...],
            preferred_element_type=jnp.float32) + b_ref[...]

    return pl.pallas_call(
        body,
        grid=(B // BLK,),
        in_specs=[
            pl.BlockSpec((BLK, K), lambda i: (i, 0)),
            pl.BlockSpec((K, OUT), lambda i: (0, 0)),
            pl.BlockSpec((1, OUT), lambda i: (0, 0)),
        ],
        out_specs=pl.BlockSpec((BLK, OUT), lambda i: (i, 0)),
        out_shape=jax.ShapeDtypeStruct((B, OUT), jnp.float32),
    )(rep, w_t, b2)


def kernel(x, lengths, table, W, b):
    B, L = x.shape
    V, D = table.shape
    x32 = x.astype(jnp.int32)
    lens = lengths.astype(jnp.int32)
    repack, tailw = _make_repack_kernel(V, D)
    tab_t = table.T
    tail_rep = table[V - tailw:].reshape(tailw * D // 128, 128)
    t_lin = repack(tab_t, tail_rep).reshape(V, D)
    pool = _make_pool_kernel(B, L, D)
    rep = pool(x32, lens, t_lin)
    return _linear(rep, W.T, b.reshape(1, -1))
